# Initial kernel scaffold; baseline (speedup 1.0000x reference)
import jax
import jax.numpy as jnp
from jax.experimental import pallas as pl


def kernel(x, tok_table, pos_table, W, b):
    raise NotImplementedError("write your pallas kernel here")


# speedup vs baseline: 2.0478x; 2.0478x over previous
import functools

import jax
import jax.numpy as jnp
from jax import lax
from jax.experimental import pallas as pl
from jax.experimental.pallas import tpu as pltpu
from jax.experimental.pallas import tpu_sc as plsc

VOCAB = 65
T = 8
BATCH = 16384
ROWS = BATCH * T
PADW = 80
GROUP = 128


def _fused_table_tc(tok_ref, pos_ref, w_ref, b_ref, out_ref):
    tok_w = jnp.dot(tok_ref[...], w_ref[...], preferred_element_type=jnp.float32)
    pos_w = jnp.dot(pos_ref[...], w_ref[...], preferred_element_type=jnp.float32)
    pos_w = pos_w + b_ref[...][None, :]
    out_ref[...] = tok_w[:, None, :] + pos_w[None, :, :]


def _make_sc_gather(num_workers):
    rows_per_w = ROWS // num_workers
    groups = rows_per_w // GROUP
    stage_words = GROUP * VOCAB
    mesh = plsc.VectorSubcoreMesh(core_axis_name="c", subcore_axis_name="s")

    @functools.partial(
        pl.kernel,
        mesh=mesh,
        out_type=jax.ShapeDtypeStruct((ROWS * VOCAB,), jnp.float32),
        scratch_types=[
            pltpu.VMEM((rows_per_w,), jnp.int32),
            pltpu.VMEM((groups, GROUP), jnp.int32),
            pltpu.VMEM((GROUP, PADW), jnp.float32),
            pltpu.VMEM((GROUP, PADW), jnp.float32),
            pltpu.VMEM((stage_words + 16,), jnp.float32),
            pltpu.VMEM((stage_words + 16,), jnp.float32),
            pltpu.SemaphoreType.DMA,
            pltpu.SemaphoreType.DMA,
            pltpu.SemaphoreType.DMA,
            pltpu.SemaphoreType.DMA,
        ],
        compiler_params=pltpu.CompilerParams(use_tc_tiling_on_sc=False),
    )
    def sc_gather(x_hbm, fused_hbm, out_hbm, x_v, idx_v, rows_a, rows_b,
                  stage_a, stage_b, sem_ga, sem_gb, sem_wa, sem_wb):
        num_cores = lax.axis_size("c")
        wid = lax.axis_index("s") * num_cores + lax.axis_index("c")
        base = wid * rows_per_w
        pltpu.sync_copy(x_hbm.at[pl.ds(base, rows_per_w)], x_v)

        tpat = lax.rem(lax.iota(jnp.int32, 16), T)

        def idx_body(g, _):
            v = x_v[pl.ds(g * 16, 16)] * T + tpat
            idx_v[g >> 3, pl.ds((g & 7) * 16, 16)] = v
            return 0

        lax.fori_loop(0, rows_per_w // 16, idx_body, 0)

        rows_bufs = (rows_a, rows_b)
        stage_bufs = (stage_a, stage_b)
        gather_sems = (sem_ga, sem_gb)
        write_sems = (sem_wa, sem_wb)
        gather_h = [None, None]
        write_h = [None, None]

        def compact(rows_v, stage_v):
            def row_body(r, _):
                for k in range(PADW // 16):
                    stage_v[pl.ds(r * VOCAB + k * 16, 16)] = rows_v[r, pl.ds(k * 16, 16)]
                return 0
            lax.fori_loop(0, GROUP, row_body, 0)

        gather_h[0] = pltpu.async_copy(
            fused_hbm.at[idx_v.at[0]], rows_bufs[0], gather_sems[0])
        for g in range(groups):
            p = g & 1
            q = 1 - p
            if g + 1 < groups:
                gather_h[q] = pltpu.async_copy(
                    fused_hbm.at[idx_v.at[g + 1]], rows_bufs[q], gather_sems[q])
            gather_h[p].wait()
            if write_h[p] is not None:
                write_h[p].wait()
            compact(rows_bufs[p], stage_bufs[p])
            write_h[p] = pltpu.async_copy(
                stage_bufs[p].at[pl.ds(0, stage_words)],
                out_hbm.at[pl.ds((base + g * GROUP) * VOCAB, stage_words)],
                write_sems[p])
        write_h[0].wait()
        write_h[1].wait()

    return sc_gather


def kernel(x, tok_table, pos_table, W, b):
    B, t = x.shape
    w_pad = jnp.pad(W, ((0, 0), (0, PADW - VOCAB)))
    b_pad = jnp.pad(b, (0, PADW - VOCAB))
    fused3 = pl.pallas_call(
        _fused_table_tc,
        out_shape=jax.ShapeDtypeStruct((VOCAB, T, PADW), jnp.float32),
    )(tok_table, pos_table, w_pad, b_pad)
    fused = fused3.reshape(VOCAB * T, PADW)

    info = plsc.get_sparse_core_info()
    num_workers = info.num_cores * info.num_subcores
    x_flat = x.reshape(-1).astype(jnp.int32)
    out = _make_sc_gather(num_workers)(x_flat, fused)
    return out.reshape(B, t, VOCAB)


# speedup vs baseline: 2.9683x; 1.4496x over previous
import functools

import jax
import jax.numpy as jnp
from jax import lax
from jax.experimental import pallas as pl
from jax.experimental.pallas import tpu as pltpu
from jax.experimental.pallas import tpu_sc as plsc

VOCAB = 65
T = 8
BATCH = 16384
ROWS = BATCH * T
PADW = 128
GROUP = 128
NBUF = 4


def _fused_table_tc(tok_ref, pos_ref, w_ref, b_ref, out_ref):
    tok_w = jnp.dot(tok_ref[...], w_ref[...], preferred_element_type=jnp.float32)
    pos_w = jnp.dot(pos_ref[...], w_ref[...], preferred_element_type=jnp.float32)
    pos_w = pos_w + b_ref[...][None, :]
    out_ref[...] = tok_w[:, None, :] + pos_w[None, :, :]


def _make_sc_gather(num_workers):
    rows_per_w = ROWS // num_workers
    groups = rows_per_w // GROUP
    mesh = plsc.VectorSubcoreMesh(core_axis_name="c", subcore_axis_name="s")

    @functools.partial(
        pl.kernel,
        mesh=mesh,
        out_type=jax.ShapeDtypeStruct((ROWS, PADW), jnp.float32),
        scratch_types=[
            pltpu.VMEM((rows_per_w,), jnp.int32),
            pltpu.VMEM((groups, GROUP), jnp.int32),
            *[pltpu.VMEM((GROUP, PADW), jnp.float32) for _ in range(NBUF)],
            *[pltpu.SemaphoreType.DMA for _ in range(NBUF)],
            *[pltpu.SemaphoreType.DMA for _ in range(NBUF)],
        ],
    )
    def sc_gather(x_hbm, fused_hbm, out_hbm, x_v, idx_v, *bufs_and_sems):
        rows_bufs = bufs_and_sems[:NBUF]
        gather_sems = bufs_and_sems[NBUF:2 * NBUF]
        write_sems = bufs_and_sems[2 * NBUF:3 * NBUF]

        num_cores = lax.axis_size("c")
        wid = lax.axis_index("s") * num_cores + lax.axis_index("c")
        base = wid * rows_per_w
        pltpu.sync_copy(x_hbm.at[pl.ds(base, rows_per_w)], x_v)

        tpat = lax.rem(lax.iota(jnp.int32, 16), T)

        def idx_body(g, _):
            v = x_v[pl.ds(g * 16, 16)] * T + tpat
            idx_v[g >> 3, pl.ds((g & 7) * 16, 16)] = v
            return 0

        lax.fori_loop(0, rows_per_w // 16, idx_body, 0)

        def gather(g, s):
            return pltpu.async_copy(
                fused_hbm.at[idx_v.at[g]], rows_bufs[s], gather_sems[s])

        gather_h = [None] * NBUF
        write_h = [None] * NBUF
        for g in range(NBUF - 1):
            gather_h[g] = gather(g, g)
        for g in range(groups):
            s = g % NBUF
            n = g + NBUF - 1
            if n < groups:
                ns = n % NBUF
                if write_h[ns] is not None:
                    write_h[ns].wait()
                    write_h[ns] = None
                gather_h[ns] = gather(n, ns)
            gather_h[s].wait()
            write_h[s] = pltpu.async_copy(
                rows_bufs[s],
                out_hbm.at[pl.ds(base + g * GROUP, GROUP)],
                write_sems[s])
        for s in range(NBUF):
            if write_h[s] is not None:
                write_h[s].wait()

    return sc_gather


def kernel(x, tok_table, pos_table, W, b):
    B, t = x.shape
    w_pad = jnp.pad(W, ((0, 0), (0, PADW - VOCAB)))
    b_pad = jnp.pad(b, (0, PADW - VOCAB))
    fused3 = pl.pallas_call(
        _fused_table_tc,
        out_shape=jax.ShapeDtypeStruct((VOCAB, T, PADW), jnp.float32),
    )(tok_table, pos_table, w_pad, b_pad)
    fused = fused3.reshape(VOCAB * T, PADW)

    info = plsc.get_sparse_core_info()
    num_workers = info.num_cores * info.num_subcores
    x_flat = x.reshape(-1).astype(jnp.int32)
    out = _make_sc_gather(num_workers)(x_flat, fused)
    return out.reshape(B, t, PADW)[:, :, :VOCAB]
